# baseline (speedup 1.0000x reference)
import functools

import jax
import jax.numpy as jnp
from jax import lax
from jax.experimental import pallas as pl
from jax.experimental.pallas import tpu as pltpu
from jax.experimental.pallas import tpu_sc as plsc

B, K, L = 32, 1024, 576
RPC = 16
NCHUNK = L // RPC
NBUF = 3

_mesh = plsc.VectorSubcoreMesh(core_axis_name="c", subcore_axis_name="s")


@functools.partial(
    pl.kernel,
    out_type=jax.ShapeDtypeStruct((B, L, K), jnp.float32),
    mesh=_mesh,
    scratch_types=[
        pltpu.VMEM((RPC, K), jnp.float32),
        pltpu.VMEM((RPC, K), jnp.float32),
        pltpu.VMEM((RPC, K), jnp.float32),
        pltpu.VMEM((RPC, K), jnp.float32),
        pltpu.VMEM((RPC, K), jnp.float32),
        pltpu.SemaphoreType.DMA,
        pltpu.SemaphoreType.DMA,
        pltpu.SemaphoreType.DMA,
        pltpu.SemaphoreType.DMA,
        pltpu.SemaphoreType.DMA,
    ],
    compiler_params=pltpu.CompilerParams(needs_layout_passes=False),
)
def _argmax_onehot(x_hbm, out_hbm, buf0, buf1, buf2, ob0, ob1,
                   si0, si1, si2, so0, so1):
    b = lax.axis_index("s") * 2 + lax.axis_index("c")
    bufs = (buf0, buf1, buf2)
    obufs = (ob0, ob1)
    in_sems = (si0, si1, si2)
    out_sems = (so0, so1)

    h_in = [
        pltpu.async_copy(x_hbm.at[b, pl.ds(c * RPC, RPC), :], bufs[c],
                         in_sems[c])
        for c in range(NBUF)
    ]

    zv = jnp.zeros((16,), jnp.float32)

    def zbody(i, _):
        def inner(j, _, i=i):
            ob0[i, pl.ds(j * 16, 16)] = zv
            ob1[i, pl.ds(j * 16, 16)] = zv
            return 0
        lax.fori_loop(0, K // 16, inner, 0, unroll=8)
        return 0

    lax.fori_loop(0, RPC, zbody, 0)

    iota = lax.iota(jnp.int32, 16)
    onev = jnp.full((16,), 1.0, jnp.float32)
    ninf = jnp.full((16,), -jnp.inf, jnp.float32)
    izero = jnp.zeros((16,), jnp.int32)
    big = jnp.full((16,), jnp.int32(1 << 30), jnp.int32)
    h_out = [None, None]
    prev = [None, None]

    for c in range(NCHUNK):
        h_in[c % NBUF].wait()
        buf = bufs[c % NBUF]

        def pbody(p, acc, buf=buf):
            def jbody(j, c2, buf=buf, p=p):
                m0, ci0, m1, ci1, jv = c2
                v0 = buf[p, pl.ds(j * 16, 16)]
                v1 = buf[p + 8, pl.ds(j * 16, 16)]
                g0 = v0 > m0
                g1 = v1 > m1
                return (
                    jnp.where(g0, v0, m0), jnp.where(g0, jv, ci0),
                    jnp.where(g1, v1, m1), jnp.where(g1, jv, ci1),
                    jv + 1,
                )

            m0, ci0, m1, ci1, _ = lax.fori_loop(
                0, K // 16, jbody, (ninf, izero, ninf, izero, izero),
                unroll=8)

            def tail(m, ci):
                hm = jnp.max(m)
                cand = jnp.where(m == hm, ci * 16 + iota, big)
                return jnp.min(cand)

            acc = jnp.where(iota == p, tail(m0, ci0), acc)
            acc = jnp.where(iota == p + 8, tail(m1, ci1), acc)
            return acc

        i0 = lax.fori_loop(0, 8, pbody, izero)

        if c + NBUF < NCHUNK:
            h_in[c % NBUF] = pltpu.async_copy(
                x_hbm.at[b, pl.ds((c + NBUF) * RPC, RPC), :], bufs[c % NBUF],
                in_sems[c % NBUF])

        obuf = obufs[c % 2]
        if h_out[c % 2] is not None:
            h_out[c % 2].wait()
            plsc.store_scatter(obuf, [iota, prev[c % 2]], zv)
        plsc.store_scatter(obuf, [iota, i0], onev)
        prev[c % 2] = i0

        h_out[c % 2] = pltpu.async_copy(
            obuf, out_hbm.at[b, pl.ds(c * RPC, RPC), :], out_sems[c % 2])

    h_out[0].wait()
    h_out[1].wait()


def kernel(x):
    xt = jnp.transpose(x, (0, 2, 1))
    ot = _argmax_onehot(xt)
    return jnp.transpose(ot, (0, 2, 1))
